# baseline (speedup 1.0000x reference)
import functools

import jax
import jax.numpy as jnp
from jax import lax
from jax.experimental import pallas as pl
from jax.experimental.pallas import tpu as pltpu
from jax.experimental.pallas import tpu_sc as plsc

N = 10000
E = 320000
G = 64
DH = 112
NW = 32
EPW = E // NW
NB = 125
BB = 80
RPT = N // 16
NR = 4

_sc_mesh = plsc.VectorSubcoreMesh(core_axis_name="c", subcore_axis_name="s")
_sc_params = pltpu.CompilerParams(needs_layout_passes=False,
                                  use_tc_tiling_on_sc=False)


@functools.partial(
    pl.kernel,
    out_type=(
        jax.ShapeDtypeStruct((NW, N), jnp.float32),
        jax.ShapeDtypeStruct((NW, N), jnp.float32),
    ),
    mesh=_sc_mesh,
    scratch_types=[
        pltpu.VMEM((EPW,), jnp.int32),
        pltpu.VMEM((EPW,), jnp.int32),
        pltpu.VMEM((N,), jnp.float32),
        pltpu.VMEM((N,), jnp.float32),
    ],
    compiler_params=_sc_params,
)
def _sc_degrees(src_hbm, dst_hbm, dsrc_out, ddst_out, src_v, dst_v, csrc_v, cdst_v):
    c = lax.axis_index("c")
    s = lax.axis_index("s")
    w = s * 2 + c
    pltpu.sync_copy(src_hbm.at[pl.ds(w * EPW, EPW)], src_v)
    pltpu.sync_copy(dst_hbm.at[pl.ds(w * EPW, EPW)], dst_v)

    def zero_body(i, carry):
        csrc_v[pl.ds(i * 16, 16)] = jnp.zeros((16,), jnp.float32)
        cdst_v[pl.ds(i * 16, 16)] = jnp.zeros((16,), jnp.float32)
        return carry

    lax.fori_loop(0, N // 16, zero_body, 0)

    ones16 = jnp.ones((16,), jnp.float32)

    def count_body(i, carry):
        si = src_v[pl.ds(i * 16, 16)]
        plsc.addupdate_scatter(csrc_v, [si], ones16)
        di = dst_v[pl.ds(i * 16, 16)]
        plsc.addupdate_scatter(cdst_v, [di], ones16)
        return carry

    lax.fori_loop(0, EPW // 16, count_body, 0)

    pltpu.sync_copy(csrc_v, dsrc_out.at[w])
    pltpu.sync_copy(cdst_v, ddst_out.at[w])


@functools.partial(
    pl.kernel,
    out_type=(
        jax.ShapeDtypeStruct((2, N, DH), jnp.float32),
        jax.ShapeDtypeStruct((2, N, DH), jnp.float32),
    ),
    mesh=_sc_mesh,
    scratch_types=[
        pltpu.VMEM((NB, BB), jnp.int32),
        pltpu.VMEM((NB, BB), jnp.int32),
        (pltpu.VMEM((BB, DH), jnp.float32),) * NR,
        pltpu.VMEM_SHARED((N, DH), jnp.float32),
        (pltpu.SemaphoreType.DMA,) * NR,
        (pltpu.SemaphoreType.DMA,) * NR,
    ],
    compiler_params=_sc_params,
)
def _sc_prop(tlo_hbm, thi_hbm, srcr_hbm, dstr_hbm, zrows_hbm, olo_hbm, ohi_hbm,
             src_v, dst_v, rows, acc_sh, gsem, ssem):
    c = lax.axis_index("c")
    s = lax.axis_index("s")
    w = s * 2 + c
    pltpu.sync_copy(srcr_hbm.at[w], src_v)
    pltpu.sync_copy(dstr_hbm.at[w], dst_v)
    base = s * RPT
    NG = (NB // NR) * NR

    for half, (tbl_hbm, out_hbm) in enumerate(((tlo_hbm, olo_hbm),
                                               (thi_hbm, ohi_hbm))):
        pltpu.sync_copy(zrows_hbm, rows[0])
        for k in range(7):
            pltpu.sync_copy(rows[0], acc_sh.at[pl.ds(base + k * BB, BB)])
        pltpu.sync_copy(rows[0].at[pl.ds(0, RPT - 7 * BB)],
                        acc_sh.at[pl.ds(base + 7 * BB, RPT - 7 * BB)])
        plsc.subcore_barrier()

        def body(i, carry):
            j = i * NR
            for r in range(NR):
                @pl.when(i > 0)
                def _():
                    pltpu.make_async_copy(
                        rows[r], acc_sh.at[dst_v.at[j + r]], ssem[r]).wait()
                pltpu.async_copy(tbl_hbm.at[src_v.at[j + r]], rows[r], gsem[r])
            for r in range(NR):
                pltpu.make_async_copy(
                    tbl_hbm.at[src_v.at[j + r]], rows[r], gsem[r]).wait()
                pltpu.async_copy(rows[r], acc_sh.at[dst_v.at[j + r]], ssem[r],
                                 add=True)
            return carry

        lax.fori_loop(0, NB // NR, body, 0)
        for r in range(NR):
            pltpu.make_async_copy(
                rows[r], acc_sh.at[dst_v.at[NG - NR + r]], ssem[r]).wait()
        for j in range(NG, NB):
            pltpu.async_copy(tbl_hbm.at[src_v.at[j]], rows[0], gsem[0]).wait()
            pltpu.sync_copy(rows[0], acc_sh.at[dst_v.at[j]], add=True)
        plsc.subcore_barrier()
        pltpu.sync_copy(acc_sh.at[pl.ds(base, RPT)],
                        out_hbm.at[c, pl.ds(base, RPT)])


def _tc_scales_body(dsrc_ref, ddst_ref, out_ref):
    deg_o = jnp.sum(dsrc_ref[...], axis=0, keepdims=True)
    deg_i = jnp.sum(ddst_ref[...], axis=0, keepdims=True)
    rout = lax.rsqrt(jnp.maximum(deg_o, 1.0))
    rin = lax.rsqrt(jnp.maximum(deg_i, 1.0))
    out_ref[...] = jnp.concatenate([rin, rout], axis=0)


def _tc_extract_body(pkt_ref, arv_ref, wp_ref, bp_ref, wa_ref, ba_ref,
                     rout_ref, olo_ref, ohi_ref):
    z = jnp.zeros((N, DH - 100), jnp.float32)
    p = jnp.maximum(
        jnp.dot(pkt_ref[...], wp_ref[...], preferred_element_type=jnp.float32)
        + bp_ref[...], 0.0)
    olo_ref[...] = jnp.concatenate([p, z], axis=1) * rout_ref[...]
    a = jnp.maximum(
        jnp.dot(arv_ref[...], wa_ref[...], preferred_element_type=jnp.float32)
        + ba_ref[...], 0.0)
    ohi_ref[...] = jnp.concatenate([a, z], axis=1) * rout_ref[...]


def _tc_mid_body(ylo_ref, yhi_ref, rin_ref, rout_ref, w0_ref, b0_ref,
                 olo_ref, ohi_ref):
    z = jnp.zeros((N, DH - 100), jnp.float32)
    yp = (ylo_ref[0] + ylo_ref[1]) * rin_ref[...]
    up = jnp.dot(yp[:, :100], w0_ref[...],
                 preferred_element_type=jnp.float32) + b0_ref[...]
    olo_ref[...] = jnp.concatenate([up, z], axis=1) * rout_ref[...]
    ya = (yhi_ref[0] + yhi_ref[1]) * rin_ref[...]
    ua = jnp.dot(ya[:, :100], w0_ref[...],
                 preferred_element_type=jnp.float32) + b0_ref[...]
    ohi_ref[...] = jnp.concatenate([ua, z], axis=1) * rout_ref[...]


def _tc_final_body(zlo_ref, zhi_ref, rin_ref, gid_ref, w1_ref, b1_ref,
                   wc_ref, bc_ref, out_ref):
    zp = (zlo_ref[0] + zlo_ref[1]) * rin_ref[...]
    za = (zhi_ref[0] + zhi_ref[1]) * rin_ref[...]
    gid = gid_ref[...]
    iot = lax.broadcasted_iota(jnp.int32, (G, N), 0)
    oh = (iot == gid).astype(jnp.float32)
    cnt = jnp.maximum(jnp.sum(oh, axis=1, keepdims=True), 1.0)
    mp = jnp.dot(oh, zp, preferred_element_type=jnp.float32)[:, :100] / cnt
    ma = jnp.dot(oh, za, preferred_element_type=jnp.float32)[:, :100] / cnt
    w1 = w1_ref[...]
    wt = wc_ref[:200]
    wb = wc_ref[200:]
    out = (jnp.dot(mp, jnp.dot(w1, wt, preferred_element_type=jnp.float32),
                   preferred_element_type=jnp.float32)
           + jnp.dot(ma, jnp.dot(w1, wb, preferred_element_type=jnp.float32),
                     preferred_element_type=jnp.float32)
           + jnp.dot(b1_ref[...], wt + wb, preferred_element_type=jnp.float32)
           + bc_ref[...])
    out_ref[...] = out


def kernel(pkt_length, arv_time, edge_index, graph_ids,
           W_pkt, b_pkt, W_arv, b_arv, W0, b0, W1, b1, Wc, bc):
    src = edge_index[0]
    dst = edge_index[1]
    srcr = src.reshape(NW, NB, BB)
    dstr = dst.reshape(NW, NB, BB)
    zrows = jnp.zeros((BB, DH), jnp.float32)

    dsrc, ddst = _sc_degrees(src, dst)

    scales = pl.pallas_call(
        _tc_scales_body,
        out_shape=jax.ShapeDtypeStruct((2, N), jnp.float32),
    )(dsrc, ddst)
    rin_col = scales[0].reshape(N, 1)
    rout_col = scales[1].reshape(N, 1)

    t1lo, t1hi = pl.pallas_call(
        _tc_extract_body,
        out_shape=(jax.ShapeDtypeStruct((N, DH), jnp.float32),
                   jax.ShapeDtypeStruct((N, DH), jnp.float32)),
    )(pkt_length, arv_time, W_pkt, b_pkt.reshape(1, -1),
      W_arv, b_arv.reshape(1, -1), rout_col)

    ylo, yhi = _sc_prop(t1lo, t1hi, srcr, dstr, zrows)

    t2lo, t2hi = pl.pallas_call(
        _tc_mid_body,
        out_shape=(jax.ShapeDtypeStruct((N, DH), jnp.float32),
                   jax.ShapeDtypeStruct((N, DH), jnp.float32)),
    )(ylo, yhi, rin_col, rout_col, W0, b0.reshape(1, -1))

    zlo, zhi = _sc_prop(t2lo, t2hi, srcr, dstr, zrows)

    out = pl.pallas_call(
        _tc_final_body,
        out_shape=jax.ShapeDtypeStruct((G, 55), jnp.float32),
    )(zlo, zhi, rin_col, graph_ids.reshape(1, N), W1, b1.reshape(1, -1),
      Wc, bc.reshape(1, -1))
    return out


# speedup vs baseline: 2.3679x; 2.3679x over previous
import functools

import jax
import jax.numpy as jnp
from jax import lax
from jax.experimental import pallas as pl
from jax.experimental.pallas import tpu as pltpu
from jax.experimental.pallas import tpu_sc as plsc

N = 10000
E = 320000
G = 64
NW = 32
EPW = E // NW
NB = 100
BB = 100
RPT = N // 16
NR = 5

_sc_mesh = plsc.VectorSubcoreMesh(core_axis_name="c", subcore_axis_name="s")
_sc_params = pltpu.CompilerParams(needs_layout_passes=False,
                                  use_tc_tiling_on_sc=False)


@functools.partial(
    pl.kernel,
    out_type=(
        jax.ShapeDtypeStruct((NW, N), jnp.float32),
        jax.ShapeDtypeStruct((NW, N), jnp.float32),
    ),
    mesh=_sc_mesh,
    scratch_types=[
        pltpu.VMEM((EPW,), jnp.int32),
        pltpu.VMEM((EPW,), jnp.int32),
        pltpu.VMEM((N,), jnp.float32),
        pltpu.VMEM((N,), jnp.float32),
    ],
    compiler_params=_sc_params,
)
def _sc_degrees(src_hbm, dst_hbm, dsrc_out, ddst_out, src_v, dst_v, csrc_v, cdst_v):
    c = lax.axis_index("c")
    s = lax.axis_index("s")
    w = s * 2 + c
    pltpu.sync_copy(src_hbm.at[pl.ds(w * EPW, EPW)], src_v)
    pltpu.sync_copy(dst_hbm.at[pl.ds(w * EPW, EPW)], dst_v)

    def zero_body(i, carry):
        csrc_v[pl.ds(i * 16, 16)] = jnp.zeros((16,), jnp.float32)
        cdst_v[pl.ds(i * 16, 16)] = jnp.zeros((16,), jnp.float32)
        return carry

    lax.fori_loop(0, N // 16, zero_body, 0)

    ones16 = jnp.ones((16,), jnp.float32)

    def count_body(i, carry):
        si = src_v[pl.ds(i * 16, 16)]
        plsc.addupdate_scatter(csrc_v, [si], ones16)
        di = dst_v[pl.ds(i * 16, 16)]
        plsc.addupdate_scatter(cdst_v, [di], ones16)
        return carry

    lax.fori_loop(0, EPW // 16, count_body, 0)

    pltpu.sync_copy(csrc_v, dsrc_out.at[w])
    pltpu.sync_copy(cdst_v, ddst_out.at[w])


@functools.partial(
    pl.kernel,
    out_type=jax.ShapeDtypeStruct((2, N, G), jnp.float32),
    mesh=_sc_mesh,
    scratch_types=[
        pltpu.VMEM((NB, BB), jnp.int32),
        pltpu.VMEM((NB, BB), jnp.int32),
        (pltpu.VMEM((BB, G), jnp.float32),) * NR,
        pltpu.VMEM_SHARED((N, G), jnp.float32),
        (pltpu.SemaphoreType.DMA,) * NR,
        (pltpu.SemaphoreType.DMA,) * NR,
    ],
    compiler_params=_sc_params,
)
def _sc_prop64(tbl_hbm, gidx_hbm, sidx_hbm, zrows_hbm, out_hbm,
               gi_v, si_v, rows, acc_sh, gsem, ssem):
    c = lax.axis_index("c")
    s = lax.axis_index("s")
    w = s * 2 + c
    pltpu.sync_copy(gidx_hbm.at[w], gi_v)
    pltpu.sync_copy(sidx_hbm.at[w], si_v)
    base = s * RPT

    pltpu.sync_copy(zrows_hbm, rows[0])
    for k in range(6):
        pltpu.sync_copy(rows[0], acc_sh.at[pl.ds(base + k * BB, BB)])
    pltpu.sync_copy(rows[0].at[pl.ds(0, RPT - 6 * BB)],
                    acc_sh.at[pl.ds(base + 6 * BB, RPT - 6 * BB)])
    plsc.subcore_barrier()

    def body(i, carry):
        j = i * NR
        for r in range(NR):
            @pl.when(i > 0)
            def _():
                pltpu.make_async_copy(
                    rows[r], acc_sh.at[si_v.at[j + r]], ssem[r]).wait()
            pltpu.async_copy(tbl_hbm.at[gi_v.at[j + r]], rows[r], gsem[r])
        for r in range(NR):
            pltpu.make_async_copy(
                tbl_hbm.at[gi_v.at[j + r]], rows[r], gsem[r]).wait()
            pltpu.async_copy(rows[r], acc_sh.at[si_v.at[j + r]], ssem[r],
                             add=True)
        return carry

    lax.fori_loop(0, NB // NR, body, 0)
    for r in range(NR):
        pltpu.make_async_copy(
            rows[r], acc_sh.at[si_v.at[NB - NR + r]], ssem[r]).wait()
    plsc.subcore_barrier()
    pltpu.sync_copy(acc_sh.at[pl.ds(base, RPT)],
                    out_hbm.at[c, pl.ds(base, RPT)])


_CN = (((0,), (0,)), ((), ()))


def _tc_extract_body(pkt_ref, arv_ref, wp_ref, bp_ref, wa_ref, ba_ref,
                     p_ref, a_ref):
    p_ref[...] = jnp.maximum(
        jnp.dot(pkt_ref[...], wp_ref[...], preferred_element_type=jnp.float32)
        + bp_ref[...], 0.0)
    a_ref[...] = jnp.maximum(
        jnp.dot(arv_ref[...], wa_ref[...], preferred_element_type=jnp.float32)
        + ba_ref[...], 0.0)


def _tc_build_body(dsrc_ref, ddst_ref, gid_ref, ones_ref,
                   rtab_ref, rin_ref, rout_ref, c0_ref):
    ones32 = ones_ref[...]
    deg_o = lax.dot_general(dsrc_ref[...], ones32, _CN,
                            preferred_element_type=jnp.float32)
    deg_i = lax.dot_general(ddst_ref[...], ones32, _CN,
                            preferred_element_type=jnp.float32)
    rout = lax.rsqrt(jnp.maximum(deg_o, 1.0))
    rin = lax.rsqrt(jnp.maximum(deg_i, 1.0))
    iot = lax.broadcasted_iota(jnp.int32, (N, G), 1)
    oh = (iot == gid_ref[...]).astype(jnp.float32)
    n_g = jnp.sum(oh, axis=0, keepdims=True)
    rtab_ref[...] = rin * oh / jnp.maximum(n_g, 1.0)
    rin_ref[...] = rin
    rout_ref[...] = rout
    c0_ref[...] = (n_g >= 1.0).astype(jnp.float32)


def _tc_mid_body(parts_ref, rin_ref, rout_ref, t2_ref, s1_ref):
    v1 = (parts_ref[0] + parts_ref[1]) * rout_ref[...]
    s1_ref[...] = jnp.sum(v1, axis=0, keepdims=True)
    t2_ref[...] = v1 * rin_ref[...]


def _tc_final_body(parts_ref, rout_ref, p_ref, a_ref, w0_ref, w1_ref,
                   wc_ref, b0_ref, b1_ref, bc_ref, s1_ref, c0_ref, out_ref):
    v2 = (parts_ref[0] + parts_ref[1]) * rout_ref[...]
    gp = lax.dot_general(v2, p_ref[...], _CN,
                         preferred_element_type=jnp.float32)
    ga = lax.dot_general(v2, a_ref[...], _CN,
                         preferred_element_type=jnp.float32)
    wt = wc_ref[:200]
    wb = wc_ref[200:]
    w1t = jnp.dot(w1_ref[...], wt, preferred_element_type=jnp.float32)
    w1b = jnp.dot(w1_ref[...], wb, preferred_element_type=jnp.float32)
    k1 = jnp.dot(w0_ref[...], w1t, preferred_element_type=jnp.float32)
    k2 = jnp.dot(w0_ref[...], w1b, preferred_element_type=jnp.float32)
    bias_mid = jnp.dot(b0_ref[...], w1t + w1b,
                       preferred_element_type=jnp.float32)
    bias_out = jnp.dot(b1_ref[...], wt + wb,
                       preferred_element_type=jnp.float32)
    out_ref[...] = (
        jnp.dot(gp, k1, preferred_element_type=jnp.float32)
        + jnp.dot(ga, k2, preferred_element_type=jnp.float32)
        + lax.dot_general(s1_ref[...], bias_mid, _CN,
                          preferred_element_type=jnp.float32)
        + lax.dot_general(c0_ref[...], bias_out, _CN,
                          preferred_element_type=jnp.float32)
        + bc_ref[...])


def kernel(pkt_length, arv_time, edge_index, graph_ids,
           W_pkt, b_pkt, W_arv, b_arv, W0, b0, W1, b1, Wc, bc):
    src = edge_index[0]
    dst = edge_index[1]
    srcr = src.reshape(NW, NB, BB)
    dstr = dst.reshape(NW, NB, BB)
    zrows = jnp.zeros((BB, G), jnp.float32)
    ones32 = jnp.ones((NW, 1), jnp.float32)

    p_feat, a_feat = pl.pallas_call(
        _tc_extract_body,
        out_shape=(jax.ShapeDtypeStruct((N, 100), jnp.float32),
                   jax.ShapeDtypeStruct((N, 100), jnp.float32)),
    )(pkt_length, arv_time, W_pkt, b_pkt.reshape(1, -1),
      W_arv, b_arv.reshape(1, -1))

    dsrc, ddst = _sc_degrees(src, dst)

    rtab, rin_col, rout_col, c0_row = pl.pallas_call(
        _tc_build_body,
        out_shape=(jax.ShapeDtypeStruct((N, G), jnp.float32),
                   jax.ShapeDtypeStruct((N, 1), jnp.float32),
                   jax.ShapeDtypeStruct((N, 1), jnp.float32),
                   jax.ShapeDtypeStruct((1, G), jnp.float32)),
    )(dsrc, ddst, graph_ids.reshape(N, 1), ones32)

    v1_parts = _sc_prop64(rtab, dstr, srcr, zrows)

    t2, s1_row = pl.pallas_call(
        _tc_mid_body,
        out_shape=(jax.ShapeDtypeStruct((N, G), jnp.float32),
                   jax.ShapeDtypeStruct((1, G), jnp.float32)),
    )(v1_parts, rin_col, rout_col)

    v2_parts = _sc_prop64(t2, dstr, srcr, zrows)

    out = pl.pallas_call(
        _tc_final_body,
        out_shape=jax.ShapeDtypeStruct((G, 55), jnp.float32),
    )(v2_parts, rout_col, p_feat, a_feat, W0, W1, Wc,
      b0.reshape(1, -1), b1.reshape(1, -1), bc.reshape(1, -1),
      s1_row, c0_row)
    return out


# speedup vs baseline: 2.4462x; 1.0331x over previous
import functools

import jax
import jax.numpy as jnp
from jax import lax
from jax.experimental import pallas as pl
from jax.experimental.pallas import tpu as pltpu
from jax.experimental.pallas import tpu_sc as plsc

N = 10000
E = 320000
G = 64
NW = 32
EPW = E // NW
NB = 100
BB = 100
RPT = N // 16
NR = 10

_sc_mesh = plsc.VectorSubcoreMesh(core_axis_name="c", subcore_axis_name="s")
_sc_params = pltpu.CompilerParams(needs_layout_passes=False,
                                  use_tc_tiling_on_sc=False)


@functools.partial(
    pl.kernel,
    out_type=(
        jax.ShapeDtypeStruct((NW, N), jnp.float32),
        jax.ShapeDtypeStruct((NW, N), jnp.float32),
    ),
    mesh=_sc_mesh,
    scratch_types=[
        pltpu.VMEM((EPW,), jnp.int32),
        pltpu.VMEM((EPW,), jnp.int32),
        pltpu.VMEM((N,), jnp.float32),
        pltpu.VMEM((N,), jnp.float32),
    ],
    compiler_params=_sc_params,
)
def _sc_degrees(src_hbm, dst_hbm, dsrc_out, ddst_out, src_v, dst_v, csrc_v, cdst_v):
    c = lax.axis_index("c")
    s = lax.axis_index("s")
    w = s * 2 + c
    pltpu.sync_copy(src_hbm.at[pl.ds(w * EPW, EPW)], src_v)
    pltpu.sync_copy(dst_hbm.at[pl.ds(w * EPW, EPW)], dst_v)

    def zero_body(i, carry):
        csrc_v[pl.ds(i * 16, 16)] = jnp.zeros((16,), jnp.float32)
        cdst_v[pl.ds(i * 16, 16)] = jnp.zeros((16,), jnp.float32)
        return carry

    lax.fori_loop(0, N // 16, zero_body, 0)

    ones16 = jnp.ones((16,), jnp.float32)

    def count_body(i, carry):
        si = src_v[pl.ds(i * 16, 16)]
        plsc.addupdate_scatter(csrc_v, [si], ones16)
        di = dst_v[pl.ds(i * 16, 16)]
        plsc.addupdate_scatter(cdst_v, [di], ones16)
        return carry

    lax.fori_loop(0, EPW // 16, count_body, 0)

    pltpu.sync_copy(csrc_v, dsrc_out.at[w])
    pltpu.sync_copy(cdst_v, ddst_out.at[w])


@functools.partial(
    pl.kernel,
    out_type=jax.ShapeDtypeStruct((2, N, G), jnp.float32),
    mesh=_sc_mesh,
    scratch_types=[
        pltpu.VMEM((NB, BB), jnp.int32),
        pltpu.VMEM((NB, BB), jnp.int32),
        (pltpu.VMEM((BB, G), jnp.float32),) * NR,
        pltpu.VMEM_SHARED((N, G), jnp.float32),
        (pltpu.SemaphoreType.DMA,) * NR,
        (pltpu.SemaphoreType.DMA,) * NR,
    ],
    compiler_params=_sc_params,
)
def _sc_prop64(tbl_hbm, gidx_hbm, sidx_hbm, zrows_hbm, out_hbm,
               gi_v, si_v, rows, acc_sh, gsem, ssem):
    c = lax.axis_index("c")
    s = lax.axis_index("s")
    w = s * 2 + c
    pltpu.sync_copy(gidx_hbm.at[w], gi_v)
    pltpu.sync_copy(sidx_hbm.at[w], si_v)
    base = s * RPT

    pltpu.sync_copy(zrows_hbm, rows[0])
    for k in range(6):
        pltpu.sync_copy(rows[0], acc_sh.at[pl.ds(base + k * BB, BB)])
    pltpu.sync_copy(rows[0].at[pl.ds(0, RPT - 6 * BB)],
                    acc_sh.at[pl.ds(base + 6 * BB, RPT - 6 * BB)])
    plsc.subcore_barrier()

    def body(i, carry):
        j = i * NR
        for r in range(NR):
            @pl.when(i > 0)
            def _():
                pltpu.make_async_copy(
                    rows[r], acc_sh.at[si_v.at[j + r]], ssem[r]).wait()
            pltpu.async_copy(tbl_hbm.at[gi_v.at[j + r]], rows[r], gsem[r])
        for r in range(NR):
            pltpu.make_async_copy(
                tbl_hbm.at[gi_v.at[j + r]], rows[r], gsem[r]).wait()
            pltpu.async_copy(rows[r], acc_sh.at[si_v.at[j + r]], ssem[r],
                             add=True)
        return carry

    lax.fori_loop(0, NB // NR, body, 0)
    for r in range(NR):
        pltpu.make_async_copy(
            rows[r], acc_sh.at[si_v.at[NB - NR + r]], ssem[r]).wait()
    plsc.subcore_barrier()
    pltpu.sync_copy(acc_sh.at[pl.ds(base, RPT)],
                    out_hbm.at[c, pl.ds(base, RPT)])


_CN = (((0,), (0,)), ((), ()))


def _tc_build_body(dsrc_ref, ddst_ref, gid_ref, ones_ref,
                   rtab_ref, rin_ref, rout_ref, c0_ref):
    ones32 = ones_ref[...]
    deg_o = lax.dot_general(dsrc_ref[...], ones32, _CN,
                            preferred_element_type=jnp.float32)
    deg_i = lax.dot_general(ddst_ref[...], ones32, _CN,
                            preferred_element_type=jnp.float32)
    rout = lax.rsqrt(jnp.maximum(deg_o, 1.0))
    rin = lax.rsqrt(jnp.maximum(deg_i, 1.0))
    iot = lax.broadcasted_iota(jnp.int32, (N, G), 1)
    oh = (iot == gid_ref[...]).astype(jnp.float32)
    n_g = jnp.sum(oh, axis=0, keepdims=True)
    rtab_ref[...] = rin * oh / jnp.maximum(n_g, 1.0)
    rin_ref[...] = rin
    rout_ref[...] = rout
    c0_ref[...] = (n_g >= 1.0).astype(jnp.float32)


def _tc_mid_body(parts_ref, rin_ref, rout_ref, t2_ref, s1_ref):
    v1 = (parts_ref[0] + parts_ref[1]) * rout_ref[...]
    s1_ref[...] = jnp.sum(v1, axis=0, keepdims=True)
    t2_ref[...] = v1 * rin_ref[...]


def _tc_final_body(parts_ref, rout_ref, pkt_ref, arv_ref,
                   wp_ref, bp_ref, wa_ref, ba_ref, w0_ref, w1_ref,
                   wc_ref, b0_ref, b1_ref, bc_ref, s1_ref, c0_ref, out_ref):
    p = jnp.maximum(
        jnp.dot(pkt_ref[...], wp_ref[...], preferred_element_type=jnp.float32)
        + bp_ref[...], 0.0)
    a = jnp.maximum(
        jnp.dot(arv_ref[...], wa_ref[...], preferred_element_type=jnp.float32)
        + ba_ref[...], 0.0)
    v2 = (parts_ref[0] + parts_ref[1]) * rout_ref[...]
    gp = lax.dot_general(v2, p, _CN, preferred_element_type=jnp.float32)
    ga = lax.dot_general(v2, a, _CN, preferred_element_type=jnp.float32)
    wt = wc_ref[:200]
    wb = wc_ref[200:]
    w1t = jnp.dot(w1_ref[...], wt, preferred_element_type=jnp.float32)
    w1b = jnp.dot(w1_ref[...], wb, preferred_element_type=jnp.float32)
    k1 = jnp.dot(w0_ref[...], w1t, preferred_element_type=jnp.float32)
    k2 = jnp.dot(w0_ref[...], w1b, preferred_element_type=jnp.float32)
    bias_mid = jnp.dot(b0_ref[...], w1t + w1b,
                       preferred_element_type=jnp.float32)
    bias_out = jnp.dot(b1_ref[...], wt + wb,
                       preferred_element_type=jnp.float32)
    out_ref[...] = (
        jnp.dot(gp, k1, preferred_element_type=jnp.float32)
        + jnp.dot(ga, k2, preferred_element_type=jnp.float32)
        + lax.dot_general(s1_ref[...], bias_mid, _CN,
                          preferred_element_type=jnp.float32)
        + lax.dot_general(c0_ref[...], bias_out, _CN,
                          preferred_element_type=jnp.float32)
        + bc_ref[...])


def kernel(pkt_length, arv_time, edge_index, graph_ids,
           W_pkt, b_pkt, W_arv, b_arv, W0, b0, W1, b1, Wc, bc):
    src = edge_index[0]
    dst = edge_index[1]
    srcr = src.reshape(NW, NB, BB)
    dstr = dst.reshape(NW, NB, BB)
    zrows = jnp.zeros((BB, G), jnp.float32)
    ones32 = jnp.ones((NW, 1), jnp.float32)

    dsrc, ddst = _sc_degrees(src, dst)

    rtab, rin_col, rout_col, c0_row = pl.pallas_call(
        _tc_build_body,
        out_shape=(jax.ShapeDtypeStruct((N, G), jnp.float32),
                   jax.ShapeDtypeStruct((N, 1), jnp.float32),
                   jax.ShapeDtypeStruct((N, 1), jnp.float32),
                   jax.ShapeDtypeStruct((1, G), jnp.float32)),
    )(dsrc, ddst, graph_ids.reshape(N, 1), ones32)

    v1_parts = _sc_prop64(rtab, dstr, srcr, zrows)

    t2, s1_row = pl.pallas_call(
        _tc_mid_body,
        out_shape=(jax.ShapeDtypeStruct((N, G), jnp.float32),
                   jax.ShapeDtypeStruct((1, G), jnp.float32)),
    )(v1_parts, rin_col, rout_col)

    v2_parts = _sc_prop64(t2, dstr, srcr, zrows)

    out = pl.pallas_call(
        _tc_final_body,
        out_shape=jax.ShapeDtypeStruct((G, 55), jnp.float32),
    )(v2_parts, rout_col, pkt_length, arv_time,
      W_pkt, b_pkt.reshape(1, -1), W_arv, b_arv.reshape(1, -1), W0, W1, Wc,
      b0.reshape(1, -1), b1.reshape(1, -1), bc.reshape(1, -1),
      s1_row, c0_row)
    return out
